# baseline (speedup 1.0000x reference)
import functools

import jax
import jax.numpy as jnp
from jax import lax
from jax.experimental import pallas as pl
from jax.experimental.pallas import tpu as pltpu
from jax.experimental.pallas import tpu_sc as plsc

BATCH = 16384
D = 32
NC = 2
NS = 16
L = 16
NW = NC * NS
CHUNK = BATCH // NW
HALF = CHUNK // 2
NSEM = 8


def _sc_body(user_emb, user_ids, item_emb, item_ids, out,
             uidx_v, iidx_v, urows_v, irows_v, preds_v, *sems):
    wid = lax.axis_index("s") * NC + lax.axis_index("c")
    base = wid * CHUNK

    pltpu.sync_copy(user_ids.at[pl.ds(base, CHUNK)], uidx_v)
    pltpu.sync_copy(item_ids.at[pl.ds(base, CHUNK)], iidx_v)

    lane = lax.broadcasted_iota(jnp.int32, (L,), 0)
    last = lane == (L - 1)

    for h in range(2):
        hbase = h * HALF

        def issue(g, carry):
            uvec = uidx_v[pl.ds(hbase + g * L, L)]
            ivec = iidx_v[pl.ds(hbase + g * L, L)]
            for j in range(L):
                i = g * L + j
                pltpu.async_copy(
                    user_emb.at[uvec[j]], urows_v.at[i], sems[(2 * j) % NSEM])
                pltpu.async_copy(
                    item_emb.at[ivec[j]], irows_v.at[i],
                    sems[(2 * j + 1) % NSEM])
            return carry

        lax.fori_loop(0, HALF // L, issue, 0)

        def drain(g, carry):
            uvec = uidx_v[pl.ds(hbase + g * L, L)]
            ivec = iidx_v[pl.ds(hbase + g * L, L)]
            for j in range(L):
                i = g * L + j
                pltpu.make_async_copy(
                    user_emb.at[uvec[j]], urows_v.at[i],
                    sems[(2 * j) % NSEM]).wait()
                pltpu.make_async_copy(
                    item_emb.at[ivec[j]], irows_v.at[i],
                    sems[(2 * j + 1) % NSEM]).wait()
            return carry

        lax.fori_loop(0, HALF // L, drain, 0)

        def row(i, carry):
            u0 = urows_v[i, pl.ds(0, L)]
            u1 = urows_v[i, pl.ds(L, L)]
            v0 = irows_v[i, pl.ds(0, L)]
            v1 = irows_v[i, pl.ds(L, L)]
            s = plsc.cumsum(u0 * v0 + u1 * v1)
            plsc.store_scatter(
                preds_v, [jnp.full((L,), hbase + i, jnp.int32)], s, mask=last)
            return carry

        lax.fori_loop(0, HALF, row, 0)

    pltpu.sync_copy(preds_v, out.at[pl.ds(base, CHUNK)])


_sc_preds = pl.kernel(
    _sc_body,
    out_type=jax.ShapeDtypeStruct((BATCH,), jnp.float32),
    mesh=plsc.VectorSubcoreMesh(core_axis_name="c", subcore_axis_name="s"),
    compiler_params=pltpu.CompilerParams(needs_layout_passes=False),
    scratch_types=[
        pltpu.VMEM((CHUNK,), jnp.int32),
        pltpu.VMEM((CHUNK,), jnp.int32),
        pltpu.VMEM((HALF, D), jnp.float32),
        pltpu.VMEM((HALF, D), jnp.float32),
        pltpu.VMEM((CHUNK,), jnp.float32),
    ] + [pltpu.SemaphoreType.DMA] * NSEM,
)


def _loss_body(p_ref, r_ref, o_ref):
    d = p_ref[...] - r_ref[...]
    o_ref[0, 0] = jnp.sum(d * d) / BATCH


_loss = pl.pallas_call(
    _loss_body,
    out_shape=jax.ShapeDtypeStruct((1, 1), jnp.float32),
    out_specs=pl.BlockSpec(memory_space=pltpu.SMEM),
)


@jax.jit
def kernel(user_ids, item_ids, ratings, user_emb, item_emb):
    preds = _sc_preds(user_emb, user_ids, item_emb, item_ids)
    loss = _loss(preds.reshape(128, 128), ratings.reshape(128, 128))[0, 0]
    return preds, loss


# speedup vs baseline: 3.1885x; 3.1885x over previous
import functools

import jax
import jax.numpy as jnp
from jax import lax
from jax.experimental import pallas as pl
from jax.experimental.pallas import tpu as pltpu
from jax.experimental.pallas import tpu_sc as plsc

BATCH = 16384
D = 32
VOCAB = 1000000
NC = 2
NS = 16
L = 16
NW = NC * NS
WCOLS = 31232
COVERED = NW * WCOLS
TAIL = VOCAB - COVERED
CCOLS = 512
NCHUNK = WCOLS // CCOLS
NBIN = NCHUNK + 1
BINCAP = 48
HITCAP = 1280
NGRP = BATCH // L
CHUNK = BATCH // NW


def _gather_body(utab, itab, uids, iids, utail, itail, urows, irows,
                 uids_v, iids_v, hu_v, hub_v, hi_v, hib_v,
                 bins_v, binb_v, ub0, ub1, ib0, ib1,
                 dep_v, cnt_s, sem0, sem1, dsem):
    wid = lax.axis_index("s") * NC + lax.axis_index("c")
    lo = wid * WCOLS
    hi = jnp.where(wid == NW - 1, jnp.int32(VOCAB), lo + WCOLS)

    pltpu.sync_copy(uids, uids_v)
    pltpu.sync_copy(iids, iids_v)

    lane = lax.broadcasted_iota(jnp.int32, (L,), 0)
    lane0 = lane == 0

    def zero_counts(t, carry):
        cnt_s[t] = 0
        return carry

    lax.fori_loop(0, 2 * NBIN, zero_counts, 0)

    def compress(g, counts):
        cu, ci = counts
        uvec = uids_v[pl.ds(g * L, L)]
        ivec = iids_v[pl.ds(g * L, L)]
        bvec = g * L + lane
        mu = (uvec >= lo) & (uvec < hi)
        mi = (ivec >= lo) & (ivec < hi)
        plsc.store_compressed(hu_v.at[pl.ds(cu, L)], uvec, mask=mu)
        plsc.store_compressed(hub_v.at[pl.ds(cu, L)], bvec, mask=mu)
        plsc.store_compressed(hi_v.at[pl.ds(ci, L)], ivec, mask=mi)
        plsc.store_compressed(hib_v.at[pl.ds(ci, L)], bvec, mask=mi)
        nu = plsc.all_reduce_population_count(mu)[0]
        ni = plsc.all_reduce_population_count(mi)[0]
        return cu + nu, ci + ni

    cu, ci = lax.fori_loop(0, NGRP, compress, (jnp.int32(0), jnp.int32(0)))

    def bin_pass(tbl, h_v, hb_v, n_hits):
        def grp(t, carry):
            hvec = h_v[pl.ds(t * L, L)]
            bvec = hb_v[pl.ds(t * L, L)]
            kvec = jnp.minimum((hvec - lo) // CCOLS, NCHUNK)
            valid = ((t * L + lane) < n_hits).astype(jnp.int32)
            for j in range(L):
                @pl.when(valid[j] == 1)
                def _():
                    bin_idx = tbl * NBIN + kvec[j]
                    slot = cnt_s[bin_idx]
                    pos = jnp.full((L,), bin_idx * BINCAP + slot)
                    plsc.store_scatter(
                        bins_v, [pos], jnp.full((L,), 0) + hvec[j],
                        mask=lane0)
                    plsc.store_scatter(
                        binb_v, [pos], jnp.full((L,), 0) + bvec[j],
                        mask=lane0)
                    cnt_s[bin_idx] = slot + 1
            return carry

        lax.fori_loop(0, (n_hits + L - 1) // L, grp, 0)

    bin_pass(0, hu_v, hub_v, cu)
    bin_pass(1, hi_v, hib_v, ci)

    d_lo = lane
    d_hi = lane + L

    def stream_chunk(tab, k, buf, sem):
        pltpu.async_copy(
            tab.at[pl.ds(0, D), pl.ds(lo + k * CCOLS, CCOLS)], buf, sem)

    def wait_chunk(tab, k, buf, sem):
        pltpu.make_async_copy(
            tab.at[pl.ds(0, D), pl.ds(lo + k * CCOLS, CCOLS)], buf,
            sem).wait()

    def extract(k, buf, tbl, out_rows):
        c0 = lo + k * CCOLS
        bin_idx = tbl * NBIN + k
        n_k = cnt_s[bin_idx]

        def grp(t, carry):
            base = bin_idx * BINCAP + t * L
            hvec = bins_v[pl.ds(base, L)]
            bvec = binb_v[pl.ds(base, L)]
            valid = ((t * L + lane) < n_k).astype(jnp.int32)
            for j in range(L):
                @pl.when(valid[j] == 1)
                def _():
                    ucol = jnp.full((L,), 0) + (hvec[j] - c0)
                    r0 = plsc.load_gather(buf, [d_lo, ucol])
                    r1 = plsc.load_gather(buf, [d_hi, ucol])
                    slot = tbl * BINCAP + (t * L + j) % BINCAP
                    dep_v[pl.ds(slot * D, L)] = r0
                    dep_v[pl.ds(slot * D + L, L)] = r1
                    pltpu.async_copy(
                        dep_v.at[pl.ds(slot * D, D)],
                        out_rows.at[pl.ds(bvec[j] * D, D)], dsem)
            return carry

        lax.fori_loop(0, (n_k + L - 1) // L, grp, 0)
        return n_k

    def drain_deposits(n):
        def one(_, carry):
            pltpu.make_async_copy(
                utail.at[pl.ds(0, D)], dep_v.at[pl.ds(0, D)], dsem).wait()
            return carry
        lax.fori_loop(0, n, one, 0)

    stream_chunk(utab, 0, ub0, sem0)
    stream_chunk(itab, 0, ib0, sem0)

    def pair(t, carry):
        k0 = 2 * t
        k1 = 2 * t + 1
        wait_chunk(utab, k0, ub0, sem0)
        wait_chunk(itab, k0, ib0, sem0)
        stream_chunk(utab, k1, ub1, sem1)
        stream_chunk(itab, k1, ib1, sem1)
        n1 = extract(k0, ub0, 0, urows)
        n2 = extract(k0, ib0, 1, irows)
        drain_deposits(n1 + n2)
        wait_chunk(utab, k1, ub1, sem1)
        wait_chunk(itab, k1, ib1, sem1)

        @pl.when(k1 + 1 < NCHUNK)
        def _():
            stream_chunk(utab, k1 + 1, ub0, sem0)
            stream_chunk(itab, k1 + 1, ib0, sem0)
        n3 = extract(k1, ub1, 0, urows)
        n4 = extract(k1, ib1, 1, irows)
        drain_deposits(n3 + n4)
        return carry

    lax.fori_loop(0, NCHUNK // 2, pair, 0)

    klast = NCHUNK - 1
    wait_chunk(utab, klast, ub0, sem0)
    wait_chunk(itab, klast, ib0, sem0)
    n1 = extract(klast, ub0, 0, urows)
    n2 = extract(klast, ib0, 1, irows)
    drain_deposits(n1 + n2)

    @pl.when(wid == NW - 1)
    def _():
        def tail_one(tbl, tail_flat, out_rows):
            bin_idx = tbl * NBIN + NCHUNK
            n_t = cnt_s[bin_idx]

            def grp(t, carry):
                base = bin_idx * BINCAP + t * L
                hvec = bins_v[pl.ds(base, L)]
                bvec = binb_v[pl.ds(base, L)]
                valid = ((t * L + lane) < n_t).astype(jnp.int32)
                for j in range(L):
                    @pl.when(valid[j] == 1)
                    def _():
                        slot = tbl * BINCAP + (t * L + j) % BINCAP
                        pltpu.sync_copy(
                            tail_flat.at[pl.ds((hvec[j] - COVERED) * D, D)],
                            dep_v.at[pl.ds(slot * D, D)])
                        pltpu.async_copy(
                            dep_v.at[pl.ds(slot * D, D)],
                            out_rows.at[pl.ds(bvec[j] * D, D)], dsem)
                return carry

            lax.fori_loop(0, (n_t + L - 1) // L, grp, 0)
            return n_t

        m1 = tail_one(0, utail, urows)
        m2 = tail_one(1, itail, irows)
        drain_deposits(m1 + m2)


_sc_gather = pl.kernel(
    _gather_body,
    out_type=(
        jax.ShapeDtypeStruct((BATCH * D,), jnp.float32),
        jax.ShapeDtypeStruct((BATCH * D,), jnp.float32),
    ),
    mesh=plsc.VectorSubcoreMesh(core_axis_name="c", subcore_axis_name="s"),
    compiler_params=pltpu.CompilerParams(needs_layout_passes=False),
    scratch_types=[
        pltpu.VMEM((BATCH,), jnp.int32),
        pltpu.VMEM((BATCH,), jnp.int32),
        pltpu.VMEM((HITCAP,), jnp.int32),
        pltpu.VMEM((HITCAP,), jnp.int32),
        pltpu.VMEM((HITCAP,), jnp.int32),
        pltpu.VMEM((HITCAP,), jnp.int32),
        pltpu.VMEM((2 * NBIN * BINCAP,), jnp.int32),
        pltpu.VMEM((2 * NBIN * BINCAP,), jnp.int32),
        pltpu.VMEM((D, CCOLS), jnp.float32),
        pltpu.VMEM((D, CCOLS), jnp.float32),
        pltpu.VMEM((D, CCOLS), jnp.float32),
        pltpu.VMEM((D, CCOLS), jnp.float32),
        pltpu.VMEM((2 * BINCAP * D,), jnp.float32),
        pltpu.SMEM((2 * NBIN,), jnp.int32),
        pltpu.SemaphoreType.DMA,
        pltpu.SemaphoreType.DMA,
        pltpu.SemaphoreType.DMA,
    ],
)


def _dot_body(urows, irows, out, u_v, i_v, preds_v, sem):
    wid = lax.axis_index("s") * NC + lax.axis_index("c")
    base = wid * CHUNK

    cu = pltpu.async_copy(urows.at[pl.ds(base * D, CHUNK * D)], u_v, sem)
    ci = pltpu.async_copy(irows.at[pl.ds(base * D, CHUNK * D)], i_v, sem)
    cu.wait()
    ci.wait()

    lane = lax.broadcasted_iota(jnp.int32, (L,), 0)
    last = lane == (L - 1)

    def row(i, carry):
        u0 = u_v[pl.ds(i * D, L)]
        u1 = u_v[pl.ds(i * D + L, L)]
        v0 = i_v[pl.ds(i * D, L)]
        v1 = i_v[pl.ds(i * D + L, L)]
        s = plsc.cumsum(u0 * v0 + u1 * v1)
        plsc.store_scatter(preds_v, [jnp.full((L,), i, jnp.int32)], s,
                           mask=last)
        return carry

    lax.fori_loop(0, CHUNK, row, 0)
    pltpu.sync_copy(preds_v, out.at[pl.ds(base, CHUNK)])


_sc_dot = pl.kernel(
    _dot_body,
    out_type=jax.ShapeDtypeStruct((BATCH,), jnp.float32),
    mesh=plsc.VectorSubcoreMesh(core_axis_name="c", subcore_axis_name="s"),
    compiler_params=pltpu.CompilerParams(needs_layout_passes=False),
    scratch_types=[
        pltpu.VMEM((CHUNK * D,), jnp.float32),
        pltpu.VMEM((CHUNK * D,), jnp.float32),
        pltpu.VMEM((CHUNK,), jnp.float32),
        pltpu.SemaphoreType.DMA,
    ],
)


def _loss_body(p_ref, r_ref, o_ref):
    d = p_ref[...] - r_ref[...]
    o_ref[0, 0] = jnp.sum(d * d) / BATCH


_loss = pl.pallas_call(
    _loss_body,
    out_shape=jax.ShapeDtypeStruct((1, 1), jnp.float32),
    out_specs=pl.BlockSpec(memory_space=pltpu.SMEM),
)


@jax.jit
def kernel(user_ids, item_ids, ratings, user_emb, item_emb):
    utail = user_emb[COVERED:, :].reshape(TAIL * D)
    itail = item_emb[COVERED:, :].reshape(TAIL * D)
    urows, irows = _sc_gather(
        user_emb.T, item_emb.T, user_ids, item_ids, utail, itail)
    preds = _sc_dot(urows, irows)
    loss = _loss(preds.reshape(128, 128), ratings.reshape(128, 128))[0, 0]
    return preds, loss
